# baseline (speedup 1.0000x reference)
import functools

import jax
import jax.numpy as jnp
from jax import lax
from jax.experimental import pallas as pl
from jax.experimental.pallas import tpu as pltpu

N = 10000
T = 12
D = 64
E = 320000


def _front_body(xe_ref, xw_ref, eeW1, eeb1, eeW2, eeb2, weW1, web1, weW2,
                web2, combWe, combWw, combb, out_ref):
    for t in range(T):
        xe = xe_ref[:, 3 * t:3 * t + 3]
        xw = xw_ref[:, 4 * t:4 * t + 4]
        he = jnp.maximum(
            jnp.dot(xe, eeW1[...], preferred_element_type=jnp.float32)
            + eeb1[...], 0.0)
        he = jnp.dot(he, eeW2[...], preferred_element_type=jnp.float32) + eeb2[...]
        hw = jnp.maximum(
            jnp.dot(xw, weW1[...], preferred_element_type=jnp.float32)
            + web1[...], 0.0)
        hw = jnp.dot(hw, weW2[...], preferred_element_type=jnp.float32) + web2[...]
        h = jnp.maximum(
            jnp.dot(he, combWe[...], preferred_element_type=jnp.float32)
            + jnp.dot(hw, combWw[...], preferred_element_type=jnp.float32)
            + combb[...], 0.0)
        out_ref[:, D * t:D * t + D] = h


def _front(Xe, Xw, ee_W1, ee_b1, ee_W2, ee_b2, we_W1, we_b1, we_W2, we_b2,
           comb_W, comb_b):
    BN = 1000
    grid = (N // BN,)
    full = lambda s: pl.BlockSpec(s, lambda i: (0,) * len(s))
    return pl.pallas_call(
        _front_body,
        grid=grid,
        in_specs=[
            pl.BlockSpec((BN, T * 3), lambda i: (i, 0)),
            pl.BlockSpec((BN, T * 4), lambda i: (i, 0)),
            full((3, 32)), full((1, 32)), full((32, D)), full((1, D)),
            full((4, 32)), full((1, 32)), full((32, D)), full((1, D)),
            full((D, D)), full((D, D)), full((1, D)),
        ],
        out_specs=pl.BlockSpec((BN, T * D), lambda i: (i, 0)),
        out_shape=jax.ShapeDtypeStruct((N, T * D), jnp.float32),
    )(Xe.reshape(N, T * 3), Xw.reshape(N, T * 4),
      ee_W1, ee_b1.reshape(1, 32), ee_W2, ee_b2.reshape(1, D),
      we_W1, we_b1.reshape(1, 32), we_W2, we_b2.reshape(1, D),
      comb_W[:D], comb_W[D:], comb_b.reshape(1, D))


def _layernorm(x, g, b):
    m = jnp.mean(x, axis=-1, keepdims=True)
    v = jnp.var(x, axis=-1, keepdims=True)
    return (x - m) / jnp.sqrt(v + 1e-5) * g + b


def _causal_conv(x, W, b):
    y = lax.conv_general_dilated(x, W, (1,), [(2, 2)],
                                 dimension_numbers=("NCH", "OIH", "NCH"))
    return y[..., :T] + b[None, :, None]


def _st_block(H, row, col, A_values, gW, gb, cfW, cfb, cgW, cgb, lng, lnb):
    outs = []
    for t in range(T):
        HWt = H[:, t, :] @ gW
        msg = A_values[:, None] * HWt[col]
        agg = jax.ops.segment_sum(msg, row, num_segments=N)
        outs.append(jax.nn.relu(agg + gb))
    Hgc = jnp.stack(outs, axis=1)
    x = jnp.transpose(Hgc, (0, 2, 1))
    f = _causal_conv(x, cfW, cfb)
    g = _causal_conv(x, cgW, cgb)
    Ht = jnp.transpose(jnp.tanh(f) * jax.nn.sigmoid(g), (0, 2, 1))
    return _layernorm(Ht + H, lng, lnb)


def kernel(X_edges, X_weather_edges, A_indices, A_values, ee_W1, ee_b1,
           ee_W2, ee_b2, we_W1, we_b1, we_W2, we_b2, comb_W, comb_b, b0_gW,
           b0_gb, b0_cfW, b0_cfb, b0_cgW, b0_cgb, b0_lng, b0_lnb, b1_gW,
           b1_gb, b1_cfW, b1_cfb, b1_cgW, b1_cgb, b1_lng, b1_lnb, ph_W1,
           ph_b1, ph_W2, ph_b2):
    H = _front(X_edges, X_weather_edges, ee_W1, ee_b1, ee_W2, ee_b2, we_W1,
               we_b1, we_W2, we_b2, comb_W, comb_b).reshape(N, T, D)
    row, col = A_indices[0], A_indices[1]
    H = _st_block(H, row, col, A_values, b0_gW, b0_gb, b0_cfW, b0_cfb,
                  b0_cgW, b0_cgb, b0_lng, b0_lnb)
    H = _st_block(H, row, col, A_values, b1_gW, b1_gb, b1_cfW, b1_cfb,
                  b1_cgW, b1_cgb, b1_lng, b1_lnb)
    Hl = H[:, -1, :]
    return jax.nn.relu(Hl @ ph_W1 + ph_b1) @ ph_W2 + ph_b2


# speedup vs baseline: 5.3872x; 5.3872x over previous
import functools

import jax
import jax.numpy as jnp
from jax import lax
from jax.experimental import pallas as pl
from jax.experimental.pallas import tpu as pltpu
from jax.experimental.pallas import tpu_sc as plsc

N = 10000
T = 12
D = 64
E = 320000

W = 128
NCHUNK = 6
NTILE = 16
BB = 128
NB = 160
E_PAD = NTILE * NB * BB
RPT = 632
NPAD = NTILE * RPT
BN = 1000


def _front_body(xe_ref, xw_ref, eeW1, eeb1, eeW2, eeb2, weW1, web1, weW2,
                web2, combWe, combWw, combb, gW, h_ref, g_ref):
    for t in range(T):
        xe = xe_ref[:, 3 * t:3 * t + 3]
        xw = xw_ref[:, 4 * t:4 * t + 4]
        he = jnp.maximum(
            jnp.dot(xe, eeW1[...], preferred_element_type=jnp.float32)
            + eeb1[...], 0.0)
        he = jnp.dot(he, eeW2[...], preferred_element_type=jnp.float32) + eeb2[...]
        hw = jnp.maximum(
            jnp.dot(xw, weW1[...], preferred_element_type=jnp.float32)
            + web1[...], 0.0)
        hw = jnp.dot(hw, weW2[...], preferred_element_type=jnp.float32) + web2[...]
        h = jnp.maximum(
            jnp.dot(he, combWe[...], preferred_element_type=jnp.float32)
            + jnp.dot(hw, combWw[...], preferred_element_type=jnp.float32)
            + combb[...], 0.0)
        h_ref[:, D * t:D * t + D] = h
        g = jnp.dot(h, gW[...], preferred_element_type=jnp.float32)
        g_ref[t // 2, :, D * (t % 2):D * (t % 2) + D] = g


def _front(Xe, Xw, ee_W1, ee_b1, ee_W2, ee_b2, we_W1, we_b1, we_W2, we_b2,
           comb_W, comb_b, gW):
    full = lambda s: pl.BlockSpec(s, lambda i: (0,) * len(s))
    return pl.pallas_call(
        _front_body,
        grid=(N // BN,),
        in_specs=[
            pl.BlockSpec((BN, T * 3), lambda i: (i, 0)),
            pl.BlockSpec((BN, T * 4), lambda i: (i, 0)),
            full((3, 32)), full((1, 32)), full((32, D)), full((1, D)),
            full((4, 32)), full((1, 32)), full((32, D)), full((1, D)),
            full((D, D)), full((D, D)), full((1, D)), full((D, D)),
        ],
        out_specs=[
            pl.BlockSpec((BN, T * D), lambda i: (i, 0)),
            pl.BlockSpec((NCHUNK, BN, W), lambda i: (0, i, 0)),
        ],
        out_shape=[
            jax.ShapeDtypeStruct((N, T * D), jnp.float32),
            jax.ShapeDtypeStruct((NCHUNK, N, W), jnp.float32),
        ],
    )(Xe.reshape(N, T * 3), Xw.reshape(N, T * 4),
      ee_W1, ee_b1.reshape(1, 32), ee_W2, ee_b2.reshape(1, D),
      we_W1, we_b1.reshape(1, 32), we_W2, we_b2.reshape(1, D),
      comb_W[:D], comb_W[D:], comb_b.reshape(1, D), gW)


def _spmm_body(G, edh, zh, out, acc, ebuf, stage, gbuf):
    core = lax.axis_index("c")
    sid = lax.axis_index("s")
    ebase = sid * NB
    rbase = sid * RPT
    for cl in range(3):
        cid = core * 3 + cl
        coff = cid * N
        pltpu.sync_copy(zh.at[pl.ds(rbase, RPT)], acc.at[pl.ds(rbase, RPT)])
        plsc.subcore_barrier()

        def batch(j, carry):
            pltpu.sync_copy(edh.at[ebase + j], ebuf.at[0])
            for m in range(8):
                sl = pl.ds(16 * m, 16)
                stage[0, sl] = ebuf[0, 0, sl] + coff
            pltpu.sync_copy(G.at[stage.at[0]], gbuf.at[0])

            def group(g, c2):
                vv = lax.bitcast_convert_type(
                    ebuf[0, 2, pl.ds(g * 16, 16)], jnp.float32)
                for k in range(16):
                    e = g * 16 + k
                    s = vv[k]
                    for m in range(8):
                        sl = pl.ds(16 * m, 16)
                        gbuf[0, e, sl] = gbuf[0, e, sl] * s
                return c2

            lax.fori_loop(0, 8, group, 0)
            pltpu.sync_copy(gbuf.at[0], acc.at[ebuf.at[0, 1]], add=True)
            return carry

        lax.fori_loop(0, NB, batch, 0)
        plsc.subcore_barrier()
        pltpu.sync_copy(acc.at[pl.ds(rbase, RPT)],
                        out.at[pl.ds(cid * NPAD + rbase, RPT)])
        plsc.subcore_barrier()


def _spmm(G_flat, edata, zeros_nw):
    mesh = plsc.VectorSubcoreMesh(core_axis_name="c", subcore_axis_name="s")
    k = pl.kernel(
        _spmm_body,
        out_type=jax.ShapeDtypeStruct((NCHUNK * NPAD, W), jnp.float32),
        mesh=mesh,
        scratch_types=[
            pltpu.VMEM_SHARED((NPAD, W), jnp.float32),
            pltpu.VMEM((2, 8, BB), jnp.int32),
            pltpu.VMEM((2, BB), jnp.int32),
            pltpu.VMEM((2, BB, W), jnp.float32),
        ],
    )
    return k(G_flat, edata, zeros_nw)


def _block_body(agg_ref, h_in, gb, cfT, cfb, cgT, cgb, lng, lnb, gWn,
                h_out, g_out):
    xs = []
    for t in range(T):
        c, jj = t // 2, t % 2
        xs.append(jnp.maximum(
            agg_ref[c, :, D * jj:D * jj + D] + gb[...], 0.0))
    zero = jnp.zeros((BN, D), jnp.float32)
    for t in range(T):
        xm2 = xs[t - 2] if t >= 2 else zero
        xm1 = xs[t - 1] if t >= 1 else zero
        xin = jnp.concatenate([xm2, xm1, xs[t]], axis=1)
        f = jnp.dot(xin, cfT[...], preferred_element_type=jnp.float32) + cfb[...]
        g = jnp.dot(xin, cgT[...], preferred_element_type=jnp.float32) + cgb[...]
        h = jnp.tanh(f) * jax.nn.sigmoid(g) + h_in[:, D * t:D * t + D]
        m = jnp.mean(h, axis=1, keepdims=True)
        v = jnp.mean((h - m) * (h - m), axis=1, keepdims=True)
        h = (h - m) * lax.rsqrt(v + 1e-5) * lng[...] + lnb[...]
        h_out[:, D * t:D * t + D] = h
        g1 = jnp.dot(h, gWn[...], preferred_element_type=jnp.float32)
        g_out[t // 2, :, D * (t % 2):D * (t % 2) + D] = g1


def _block_tc(agg, H, gb, cfT, cfb, cgT, cgb, lng, lnb, gWn):
    full = lambda s: pl.BlockSpec(s, lambda i: (0,) * len(s))
    return pl.pallas_call(
        _block_body,
        grid=(N // BN,),
        in_specs=[
            pl.BlockSpec((NCHUNK, BN, W), lambda i: (0, i, 0)),
            pl.BlockSpec((BN, T * D), lambda i: (i, 0)),
            full((1, D)), full((3 * D, D)), full((1, D)),
            full((3 * D, D)), full((1, D)), full((1, D)), full((1, D)),
            full((D, D)),
        ],
        out_specs=[
            pl.BlockSpec((BN, T * D), lambda i: (i, 0)),
            pl.BlockSpec((NCHUNK, BN, W), lambda i: (0, i, 0)),
        ],
        out_shape=[
            jax.ShapeDtypeStruct((N, T * D), jnp.float32),
            jax.ShapeDtypeStruct((NCHUNK, N, W), jnp.float32),
        ],
    )(agg, H, gb, cfT, cfb, cgT, cgb, lng, lnb, gWn)


def _tail_body(agg_ref, h_in, gb, cfT, cfb, cgT, cgb, lng, lnb, phW1, phb1,
               phW2, phb2, out_ref):
    xs = []
    for t in (9, 10, 11):
        c, jj = t // 2 - 4, t % 2
        xs.append(jnp.maximum(
            agg_ref[c, :, D * jj:D * jj + D] + gb[...], 0.0))
    xin = jnp.concatenate(xs, axis=1)
    f = jnp.dot(xin, cfT[...], preferred_element_type=jnp.float32) + cfb[...]
    g = jnp.dot(xin, cgT[...], preferred_element_type=jnp.float32) + cgb[...]
    h = jnp.tanh(f) * jax.nn.sigmoid(g) + h_in[:, D * 11:D * 12]
    m = jnp.mean(h, axis=1, keepdims=True)
    v = jnp.mean((h - m) * (h - m), axis=1, keepdims=True)
    h = (h - m) * lax.rsqrt(v + 1e-5) * lng[...] + lnb[...]
    p = jnp.maximum(
        jnp.dot(h, phW1[...], preferred_element_type=jnp.float32)
        + phb1[...], 0.0)
    out_ref[...] = (jnp.dot(p, phW2[...], preferred_element_type=jnp.float32)
                    + phb2[...])


def _tail_tc(agg, H, gb, cfT, cfb, cgT, cgb, lng, lnb, phW1, phb1, phW2p,
             phb2p):
    full = lambda s: pl.BlockSpec(s, lambda i: (0,) * len(s))
    return pl.pallas_call(
        _tail_body,
        grid=(N // BN,),
        in_specs=[
            pl.BlockSpec((2, BN, W), lambda i: (0, i, 0)),
            pl.BlockSpec((BN, T * D), lambda i: (i, 0)),
            full((1, D)), full((3 * D, D)), full((1, D)),
            full((3 * D, D)), full((1, D)), full((1, D)), full((1, D)),
            full((D, 32)), full((1, 32)), full((32, 128)), full((1, 128)),
        ],
        out_specs=pl.BlockSpec((BN, 128), lambda i: (i, 0)),
        out_shape=jax.ShapeDtypeStruct((N, 128), jnp.float32),
    )(agg[4:], H, gb, cfT, cfb, cgT, cgb, lng, lnb, phW1, phb1, phW2p,
      phb2p)


def _conv_t(Wc):
    return jnp.concatenate([Wc[:, :, k].T for k in range(3)], axis=0)


def kernel(X_edges, X_weather_edges, A_indices, A_values, ee_W1, ee_b1,
           ee_W2, ee_b2, we_W1, we_b1, we_W2, we_b2, comb_W, comb_b, b0_gW,
           b0_gb, b0_cfW, b0_cfb, b0_cgW, b0_cgb, b0_lng, b0_lnb, b1_gW,
           b1_gb, b1_cfW, b1_cfb, b1_cgW, b1_cgb, b1_lng, b1_lnb, ph_W1,
           ph_b1, ph_W2, ph_b2):
    f32 = jnp.float32
    row, col = A_indices[0], A_indices[1]
    pad = E_PAD - E
    colp = jnp.concatenate([col, jnp.zeros((pad,), jnp.int32)]).reshape(
        NTILE * NB, BB)
    rowp = jnp.concatenate([row, jnp.zeros((pad,), jnp.int32)]).reshape(
        NTILE * NB, BB)
    valp = jnp.concatenate([A_values, jnp.zeros((pad,), f32)]).reshape(
        NTILE * NB, BB)
    valbits = lax.bitcast_convert_type(valp, jnp.int32)
    edata = (jnp.zeros((NTILE * NB, 8, BB), jnp.int32)
             .at[:, 0, :].set(colp)
             .at[:, 1, :].set(rowp)
             .at[:, 2, :].set(valbits))
    zeros_nw = jnp.zeros((NPAD, W), f32)

    H0, G0 = _front(X_edges, X_weather_edges, ee_W1, ee_b1, ee_W2, ee_b2,
                    we_W1, we_b1, we_W2, we_b2, comb_W, comb_b, b0_gW)
    agg0 = _spmm(G0.reshape(NCHUNK * N, W), edata,
                 zeros_nw).reshape(NCHUNK, NPAD, W)[:, :N, :]
    H1, G1 = _block_tc(agg0, H0, b0_gb.reshape(1, D), _conv_t(b0_cfW),
                       b0_cfb.reshape(1, D), _conv_t(b0_cgW),
                       b0_cgb.reshape(1, D), b0_lng.reshape(1, D),
                       b0_lnb.reshape(1, D), b1_gW)
    agg1 = _spmm(G1.reshape(NCHUNK * N, W), edata,
                 zeros_nw).reshape(NCHUNK, NPAD, W)[:, :N, :]
    phW2p = jnp.zeros((32, 128), f32).at[:, 0].set(ph_W2[:, 0])
    phb2p = jnp.zeros((1, 128), f32).at[0, 0].set(ph_b2[0])
    outp = _tail_tc(agg1, H1, b1_gb.reshape(1, D), _conv_t(b1_cfW),
                    b1_cfb.reshape(1, D), _conv_t(b1_cgW),
                    b1_cgb.reshape(1, D), b1_lng.reshape(1, D),
                    b1_lnb.reshape(1, D), ph_W1, ph_b1.reshape(1, 32),
                    phW2p, phb2p)
    return outp[:, :1]


# speedup vs baseline: 6.1972x; 1.1504x over previous
import functools

import jax
import jax.numpy as jnp
from jax import lax
from jax.experimental import pallas as pl
from jax.experimental.pallas import tpu as pltpu
from jax.experimental.pallas import tpu_sc as plsc

N = 10000
T = 12
D = 64
E = 320000

W = 128
NCHUNK = 6
NTILE = 16
BB = 128
NB = 160
E_PAD = NTILE * NB * BB
RPT = 632
NPAD = NTILE * RPT
BN = 1000


def _front_body(xe_ref, xw_ref, eeW1, eeb1, eeW2, eeb2, weW1, web1, weW2,
                web2, combWe, combWw, combb, gW, h_ref, g_ref):
    for t in range(T):
        xe = xe_ref[:, 3 * t:3 * t + 3]
        xw = xw_ref[:, 4 * t:4 * t + 4]
        he = jnp.maximum(
            jnp.dot(xe, eeW1[...], preferred_element_type=jnp.float32)
            + eeb1[...], 0.0)
        he = jnp.dot(he, eeW2[...], preferred_element_type=jnp.float32) + eeb2[...]
        hw = jnp.maximum(
            jnp.dot(xw, weW1[...], preferred_element_type=jnp.float32)
            + web1[...], 0.0)
        hw = jnp.dot(hw, weW2[...], preferred_element_type=jnp.float32) + web2[...]
        h = jnp.maximum(
            jnp.dot(he, combWe[...], preferred_element_type=jnp.float32)
            + jnp.dot(hw, combWw[...], preferred_element_type=jnp.float32)
            + combb[...], 0.0)
        h_ref[:, D * t:D * t + D] = h
        g = jnp.dot(h, gW[...], preferred_element_type=jnp.float32)
        g_ref[t // 2, :, D * (t % 2):D * (t % 2) + D] = g


def _front(Xe, Xw, ee_W1, ee_b1, ee_W2, ee_b2, we_W1, we_b1, we_W2, we_b2,
           comb_W, comb_b, gW):
    full = lambda s: pl.BlockSpec(s, lambda i: (0,) * len(s))
    return pl.pallas_call(
        _front_body,
        grid=(N // BN,),
        in_specs=[
            pl.BlockSpec((BN, T * 3), lambda i: (i, 0)),
            pl.BlockSpec((BN, T * 4), lambda i: (i, 0)),
            full((3, 32)), full((1, 32)), full((32, D)), full((1, D)),
            full((4, 32)), full((1, 32)), full((32, D)), full((1, D)),
            full((D, D)), full((D, D)), full((1, D)), full((D, D)),
        ],
        out_specs=[
            pl.BlockSpec((BN, T * D), lambda i: (i, 0)),
            pl.BlockSpec((NCHUNK, BN, W), lambda i: (0, i, 0)),
        ],
        out_shape=[
            jax.ShapeDtypeStruct((N, T * D), jnp.float32),
            jax.ShapeDtypeStruct((NCHUNK, N, W), jnp.float32),
        ],
    )(Xe.reshape(N, T * 3), Xw.reshape(N, T * 4),
      ee_W1, ee_b1.reshape(1, 32), ee_W2, ee_b2.reshape(1, D),
      we_W1, we_b1.reshape(1, 32), we_W2, we_b2.reshape(1, D),
      comb_W[:D], comb_W[D:], comb_b.reshape(1, D), gW)


def _spmm_body(G, edh, zh, out, acc, ebuf, gbuf, semg, semsc, seme):
    core = lax.axis_index("c")
    sid = lax.axis_index("s")
    ebase = sid * NB
    rbase = sid * RPT

    def eload(cid, j, slot):
        return pltpu.async_copy(edh.at[cid, ebase + j], ebuf.at[slot],
                                seme.at[slot])

    def ewait(cid, j, slot):
        pltpu.make_async_copy(edh.at[cid, ebase + j], ebuf.at[slot],
                              seme.at[slot]).wait()

    def gfire(b, slot):
        return pltpu.async_copy(G.at[ebuf.at[slot, 0]], gbuf.at[b],
                                semg.at[b])

    def gwait(b, slot):
        pltpu.make_async_copy(G.at[ebuf.at[slot, 0]], gbuf.at[b],
                              semg.at[b]).wait()

    def sfire(b, slot):
        return pltpu.async_copy(gbuf.at[b], acc.at[ebuf.at[slot, 1]],
                                semsc.at[b], add=True)

    def swait(b, slot):
        pltpu.make_async_copy(gbuf.at[b], acc.at[ebuf.at[slot, 1]],
                              semsc.at[b]).wait()

    for cl in range(3):
        cid = core * 3 + cl
        pltpu.sync_copy(zh.at[pl.ds(rbase, RPT)], acc.at[pl.ds(rbase, RPT)])
        plsc.subcore_barrier()

        eload(cid, 0, 0)
        eload(cid, 1, 1)
        ewait(cid, 0, 0)
        gfire(0, 0)

        def step4(i, carry):
            for k in range(4):
                j = i * 4 + k
                b = k % 2
                es = k
                gwait(b, es)

                def group(g, c2):
                    vv = lax.bitcast_convert_type(
                        ebuf[es, 2, pl.ds(g * 16, 16)], jnp.float32)
                    for kk in range(16):
                        e = g * 16 + kk
                        s = vv[kk]
                        for m in range(8):
                            sl = pl.ds(16 * m, 16)
                            gbuf[b, e, sl] = gbuf[b, e, sl] * s
                    return c2

                lax.fori_loop(0, 8, group, 0)
                sfire(b, es)
                b1, e1 = (k + 1) % 2, (k + 1) % 4

                @pl.when(j < NB - 1)
                def _():
                    ewait(cid, j + 1, e1)

                    @pl.when(j > 0)
                    def _():
                        swait(b1, e1)

                    gfire(b1, e1)

                @pl.when(j < NB - 2)
                def _():
                    eload(cid, j + 2, (k + 2) % 4)
            return carry

        lax.fori_loop(0, NB // 4, step4, 0)
        for b in range(2):
            pltpu.make_async_copy(zh.at[pl.ds(0, BB)], gbuf.at[b],
                                  semsc.at[b]).wait()
        plsc.subcore_barrier()
        pltpu.sync_copy(acc.at[pl.ds(rbase, RPT)],
                        out.at[pl.ds(cid * NPAD + rbase, RPT)])
        plsc.subcore_barrier()


def _spmm(G_flat, edata, zeros_nw):
    mesh = plsc.VectorSubcoreMesh(core_axis_name="c", subcore_axis_name="s")
    k = pl.kernel(
        _spmm_body,
        out_type=jax.ShapeDtypeStruct((NCHUNK * NPAD, W), jnp.float32),
        mesh=mesh,
        scratch_types=[
            pltpu.VMEM_SHARED((NPAD, W), jnp.float32),
            pltpu.VMEM((4, 8, BB), jnp.int32),
            pltpu.VMEM((2, BB, W), jnp.float32),
            pltpu.SemaphoreType.DMA((2,)),
            pltpu.SemaphoreType.DMA((2,)),
            pltpu.SemaphoreType.DMA((4,)),
        ],
    )
    return k(G_flat, edata, zeros_nw)


def _block_body(agg_ref, h_in, gb, cfT, cfb, cgT, cgb, lng, lnb, gWn,
                h_out, g_out):
    xs = []
    for t in range(T):
        c, jj = t // 2, t % 2
        xs.append(jnp.maximum(
            agg_ref[c, :, D * jj:D * jj + D] + gb[...], 0.0))
    zero = jnp.zeros((BN, D), jnp.float32)
    for t in range(T):
        xm2 = xs[t - 2] if t >= 2 else zero
        xm1 = xs[t - 1] if t >= 1 else zero
        xin = jnp.concatenate([xm2, xm1, xs[t]], axis=1)
        f = jnp.dot(xin, cfT[...], preferred_element_type=jnp.float32) + cfb[...]
        g = jnp.dot(xin, cgT[...], preferred_element_type=jnp.float32) + cgb[...]
        h = jnp.tanh(f) * jax.nn.sigmoid(g) + h_in[:, D * t:D * t + D]
        m = jnp.mean(h, axis=1, keepdims=True)
        v = jnp.mean((h - m) * (h - m), axis=1, keepdims=True)
        h = (h - m) * lax.rsqrt(v + 1e-5) * lng[...] + lnb[...]
        h_out[:, D * t:D * t + D] = h
        g1 = jnp.dot(h, gWn[...], preferred_element_type=jnp.float32)
        g_out[t // 2, :, D * (t % 2):D * (t % 2) + D] = g1


def _block_tc(agg, H, gb, cfT, cfb, cgT, cgb, lng, lnb, gWn):
    full = lambda s: pl.BlockSpec(s, lambda i: (0,) * len(s))
    return pl.pallas_call(
        _block_body,
        grid=(N // BN,),
        in_specs=[
            pl.BlockSpec((NCHUNK, BN, W), lambda i: (0, i, 0)),
            pl.BlockSpec((BN, T * D), lambda i: (i, 0)),
            full((1, D)), full((3 * D, D)), full((1, D)),
            full((3 * D, D)), full((1, D)), full((1, D)), full((1, D)),
            full((D, D)),
        ],
        out_specs=[
            pl.BlockSpec((BN, T * D), lambda i: (i, 0)),
            pl.BlockSpec((NCHUNK, BN, W), lambda i: (0, i, 0)),
        ],
        out_shape=[
            jax.ShapeDtypeStruct((N, T * D), jnp.float32),
            jax.ShapeDtypeStruct((NCHUNK, N, W), jnp.float32),
        ],
    )(agg, H, gb, cfT, cfb, cgT, cgb, lng, lnb, gWn)


def _tail_body(agg_ref, h_in, gb, cfT, cfb, cgT, cgb, lng, lnb, phW1, phb1,
               phW2, phb2, out_ref):
    xs = []
    for t in (9, 10, 11):
        c, jj = t // 2 - 4, t % 2
        xs.append(jnp.maximum(
            agg_ref[c, :, D * jj:D * jj + D] + gb[...], 0.0))
    xin = jnp.concatenate(xs, axis=1)
    f = jnp.dot(xin, cfT[...], preferred_element_type=jnp.float32) + cfb[...]
    g = jnp.dot(xin, cgT[...], preferred_element_type=jnp.float32) + cgb[...]
    h = jnp.tanh(f) * jax.nn.sigmoid(g) + h_in[:, D * 11:D * 12]
    m = jnp.mean(h, axis=1, keepdims=True)
    v = jnp.mean((h - m) * (h - m), axis=1, keepdims=True)
    h = (h - m) * lax.rsqrt(v + 1e-5) * lng[...] + lnb[...]
    p = jnp.maximum(
        jnp.dot(h, phW1[...], preferred_element_type=jnp.float32)
        + phb1[...], 0.0)
    out_ref[...] = (jnp.dot(p, phW2[...], preferred_element_type=jnp.float32)
                    + phb2[...])


def _tail_tc(agg, H, gb, cfT, cfb, cgT, cgb, lng, lnb, phW1, phb1, phW2p,
             phb2p):
    full = lambda s: pl.BlockSpec(s, lambda i: (0,) * len(s))
    return pl.pallas_call(
        _tail_body,
        grid=(N // BN,),
        in_specs=[
            pl.BlockSpec((2, BN, W), lambda i: (0, i, 0)),
            pl.BlockSpec((BN, T * D), lambda i: (i, 0)),
            full((1, D)), full((3 * D, D)), full((1, D)),
            full((3 * D, D)), full((1, D)), full((1, D)), full((1, D)),
            full((D, 32)), full((1, 32)), full((32, 128)), full((1, 128)),
        ],
        out_specs=pl.BlockSpec((BN, 128), lambda i: (i, 0)),
        out_shape=jax.ShapeDtypeStruct((N, 128), jnp.float32),
    )(agg[4:], H, gb, cfT, cfb, cgT, cgb, lng, lnb, phW1, phb1, phW2p,
      phb2p)


def _conv_t(Wc):
    return jnp.concatenate([Wc[:, :, k].T for k in range(3)], axis=0)


def kernel(X_edges, X_weather_edges, A_indices, A_values, ee_W1, ee_b1,
           ee_W2, ee_b2, we_W1, we_b1, we_W2, we_b2, comb_W, comb_b, b0_gW,
           b0_gb, b0_cfW, b0_cfb, b0_cgW, b0_cgb, b0_lng, b0_lnb, b1_gW,
           b1_gb, b1_cfW, b1_cfb, b1_cgW, b1_cgb, b1_lng, b1_lnb, ph_W1,
           ph_b1, ph_W2, ph_b2):
    f32 = jnp.float32
    row, col = A_indices[0], A_indices[1]
    pad = E_PAD - E
    colp = jnp.concatenate([col, jnp.zeros((pad,), jnp.int32)]).reshape(
        NTILE * NB, BB)
    rowp = jnp.concatenate([row, jnp.zeros((pad,), jnp.int32)]).reshape(
        NTILE * NB, BB)
    valp = jnp.concatenate([A_values, jnp.zeros((pad,), f32)]).reshape(
        NTILE * NB, BB)
    valbits = lax.bitcast_convert_type(valp, jnp.int32)
    col6 = colp[None] + (jnp.arange(NCHUNK, dtype=jnp.int32) * N)[:, None,
                                                                  None]
    edata = (jnp.zeros((NCHUNK, NTILE * NB, 8, BB), jnp.int32)
             .at[:, :, 0, :].set(col6)
             .at[:, :, 1, :].set(rowp[None])
             .at[:, :, 2, :].set(valbits[None]))
    zeros_nw = jnp.zeros((NPAD, W), f32)

    H0, G0 = _front(X_edges, X_weather_edges, ee_W1, ee_b1, ee_W2, ee_b2,
                    we_W1, we_b1, we_W2, we_b2, comb_W, comb_b, b0_gW)
    agg0 = _spmm(G0.reshape(NCHUNK * N, W), edata,
                 zeros_nw).reshape(NCHUNK, NPAD, W)[:, :N, :]
    H1, G1 = _block_tc(agg0, H0, b0_gb.reshape(1, D), _conv_t(b0_cfW),
                       b0_cfb.reshape(1, D), _conv_t(b0_cgW),
                       b0_cgb.reshape(1, D), b0_lng.reshape(1, D),
                       b0_lnb.reshape(1, D), b1_gW)
    agg1 = _spmm(G1.reshape(NCHUNK * N, W), edata,
                 zeros_nw).reshape(NCHUNK, NPAD, W)[:, :N, :]
    phW2p = jnp.zeros((32, 128), f32).at[:, 0].set(ph_W2[:, 0])
    phb2p = jnp.zeros((1, 128), f32).at[0, 0].set(ph_b2[0])
    outp = _tail_tc(agg1, H1, b1_gb.reshape(1, D), _conv_t(b1_cfW),
                    b1_cfb.reshape(1, D), _conv_t(b1_cgW),
                    b1_cgb.reshape(1, D), b1_lng.reshape(1, D),
                    b1_lnb.reshape(1, D), ph_W1, ph_b1.reshape(1, 32),
                    phW2p, phb2p)
    return outp[:, :1]
